# Initial kernel scaffold; baseline (speedup 1.0000x reference)
import jax
import jax.numpy as jnp
from jax.experimental import pallas as pl


def kernel(tokens, vocab_map):
    raise NotImplementedError("write your pallas kernel here")


# speedup vs baseline: 139.2510x; 139.2510x over previous
import functools

import jax
import jax.numpy as jnp
from jax import lax
from jax.experimental import pallas as pl
from jax.experimental.pallas import tpu as pltpu
from jax.experimental.pallas import tpu_sc as plsc

BATCH = 16384
SEQ_LEN = 200
VOCAB = 1000000
N = BATCH * SEQ_LEN

_info = plsc.get_sparse_core_info()
NC = _info.num_cores
NS = _info.num_subcores
NW = NC * NS
N_PER_W = N // NW
CHUNK = 25600
NCHUNK = N_PER_W // CHUNK


def _make_kernel():
    mesh = plsc.VectorSubcoreMesh(core_axis_name="c", subcore_axis_name="s")

    @functools.partial(
        pl.kernel,
        mesh=mesh,
        out_type=jax.ShapeDtypeStruct((N,), jnp.int32),
        scratch_types=[
            pltpu.VMEM((CHUNK,), jnp.int32),
            pltpu.VMEM((CHUNK,), jnp.int32),
            pltpu.SemaphoreType.DMA,
        ],
    )
    def gather_kernel(tok_hbm, vocab_hbm, out_hbm, idx_v, val_v, sem):
        wid = lax.axis_index("s") * NC + lax.axis_index("c")
        for k in range(NCHUNK):
            base = wid * N_PER_W + k * CHUNK
            pltpu.sync_copy(tok_hbm.at[pl.ds(base, CHUNK)], idx_v)
            pltpu.async_copy(vocab_hbm.at[idx_v], val_v, sem).wait()
            pltpu.sync_copy(val_v, out_hbm.at[pl.ds(base, CHUNK)])

    return gather_kernel


_gather = _make_kernel()


def kernel(tokens, vocab_map):
    flat = tokens.reshape(N)
    out = _gather(flat, vocab_map)
    return out.reshape(BATCH, SEQ_LEN)


# speedup vs baseline: 229.6549x; 1.6492x over previous
import functools

import jax
import jax.numpy as jnp
from jax import lax
from jax.experimental import pallas as pl
from jax.experimental.pallas import tpu as pltpu
from jax.experimental.pallas import tpu_sc as plsc

BATCH = 16384
SEQ_LEN = 200
VOCAB = 1000000
N = BATCH * SEQ_LEN

_info = plsc.get_sparse_core_info()
NC = _info.num_cores
NS = _info.num_subcores
NW = NC * NS
N_PER_W = N // NW
CHUNK = 12800
NCHUNK = N_PER_W // CHUNK
SLICE = (VOCAB // NS) // 8 * 8
TAIL = VOCAB - NS * SLICE


def _make_kernel():
    mesh = plsc.VectorSubcoreMesh(core_axis_name="c", subcore_axis_name="s")

    @functools.partial(
        pl.kernel,
        mesh=mesh,
        out_type=jax.ShapeDtypeStruct((N,), jnp.int32),
        scratch_types=[
            pltpu.VMEM_SHARED((VOCAB,), jnp.int32),
            pltpu.VMEM((CHUNK,), jnp.int32),
            pltpu.VMEM((CHUNK,), jnp.int32),
            pltpu.VMEM((CHUNK,), jnp.int32),
            pltpu.VMEM((CHUNK,), jnp.int32),
            pltpu.SemaphoreType.DMA,
            pltpu.SemaphoreType.DMA,
            pltpu.SemaphoreType.DMA,
            pltpu.SemaphoreType.DMA,
            pltpu.SemaphoreType.DMA,
        ],
    )
    def gather_kernel(tok_hbm, vocab_hbm, out_hbm, table_sh,
                      idx0, idx1, val0, val1,
                      sem_i0, sem_i1, sem_o0, sem_o1, sem_g):
        sid = lax.axis_index("s")
        wid = sid * NC + lax.axis_index("c")
        base = wid * N_PER_W
        idxs = (idx0, idx1)
        vals = (val0, val1)
        sems_i = (sem_i0, sem_i1)
        sems_o = (sem_o0, sem_o1)

        in_cps = [None, None]
        in_cps[0] = pltpu.make_async_copy(
            tok_hbm.at[pl.ds(base, CHUNK)], idx0, sem_i0)
        in_cps[0].start()

        n_full, last = divmod(SLICE, CHUNK)
        for j in range(n_full + (1 if last else 0)):
            ln = CHUNK if j < n_full else last
            off = sid * SLICE + j * CHUNK
            pltpu.sync_copy(vocab_hbm.at[pl.ds(off, ln)], val0.at[pl.ds(0, ln)])
            pltpu.sync_copy(val0.at[pl.ds(0, ln)], table_sh.at[pl.ds(off, ln)])

        @pl.when(sid == 0)
        def _():
            off = NS * SLICE
            pltpu.sync_copy(vocab_hbm.at[pl.ds(off, TAIL)],
                            val0.at[pl.ds(0, TAIL)])
            pltpu.sync_copy(val0.at[pl.ds(0, TAIL)],
                            table_sh.at[pl.ds(off, TAIL)])

        plsc.subcore_barrier()

        out_cps = [None, None]
        for k in range(NCHUNK):
            b = k % 2
            if k + 1 < NCHUNK:
                cp = pltpu.make_async_copy(
                    tok_hbm.at[pl.ds(base + (k + 1) * CHUNK, CHUNK)],
                    idxs[1 - b], sems_i[1 - b])
                cp.start()
                in_cps[1 - b] = cp
            in_cps[b].wait()
            if out_cps[b] is not None:
                out_cps[b].wait()
            pltpu.async_copy(table_sh.at[idxs[b]], vals[b], sem_g).wait()
            cp = pltpu.make_async_copy(
                vals[b], out_hbm.at[pl.ds(base + k * CHUNK, CHUNK)], sems_o[b])
            cp.start()
            out_cps[b] = cp
        for cp in out_cps:
            if cp is not None:
                cp.wait()

    return gather_kernel


_gather = _make_kernel()


def kernel(tokens, vocab_map):
    flat = tokens.reshape(N)
    out = _gather(flat, vocab_map)
    return out.reshape(BATCH, SEQ_LEN)


# speedup vs baseline: 324.8478x; 1.4145x over previous
import functools

import jax
import jax.numpy as jnp
from jax import lax
from jax.experimental import pallas as pl
from jax.experimental.pallas import tpu as pltpu
from jax.experimental.pallas import tpu_sc as plsc

BATCH = 16384
SEQ_LEN = 200
LANE = 128
REM = SEQ_LEN - LANE
VOCAB = 1000000

_info = plsc.get_sparse_core_info()
NC = _info.num_cores
NS = _info.num_subcores
NW = NC * NS
ROWS_PER_W = BATCH // NW
CROWS = 32
NCHUNK = ROWS_PER_W // CROWS
GROUP = 8
STG = 12800
SLICE = (VOCAB // NS) // 8 * 8
TAIL = VOCAB - NS * SLICE


def _make_kernel():
    mesh = plsc.VectorSubcoreMesh(core_axis_name="c", subcore_axis_name="s")

    @functools.partial(
        pl.kernel,
        mesh=mesh,
        out_type=jax.ShapeDtypeStruct((BATCH, SEQ_LEN), jnp.int32),
        scratch_types=[
            pltpu.VMEM_SHARED((VOCAB,), jnp.int32),
            pltpu.VMEM((STG,), jnp.int32),
            [pltpu.VMEM((CROWS, LANE), jnp.int32) for _ in range(2)],
            [pltpu.VMEM((CROWS, LANE), jnp.int32) for _ in range(2)],
            [pltpu.VMEM((CROWS, REM), jnp.int32) for _ in range(2)],
            [pltpu.VMEM((CROWS, REM), jnp.int32) for _ in range(2)],
            [pltpu.SemaphoreType.DMA for _ in range(2)],
            [pltpu.SemaphoreType.DMA for _ in range(2)],
            pltpu.SemaphoreType.DMA,
        ],
    )
    def gather_kernel(tok_hbm, vocab_hbm, out_hbm, table_sh, stg,
                      idxL, valL, idxR, valR, sems_i, sems_o, sem_g):
        sid = lax.axis_index("s")
        wid = sid * NC + lax.axis_index("c")
        row_base = wid * ROWS_PER_W

        def in_cps(k, b):
            r0 = row_base + k * CROWS
            return (
                pltpu.make_async_copy(
                    tok_hbm.at[pl.ds(r0, CROWS), pl.ds(0, LANE)],
                    idxL[b], sems_i[b]),
                pltpu.make_async_copy(
                    tok_hbm.at[pl.ds(r0, CROWS), pl.ds(LANE, REM)],
                    idxR[b], sems_i[b]),
            )

        def out_cps(k, b):
            r0 = row_base + k * CROWS
            return (
                pltpu.make_async_copy(
                    valL[b], out_hbm.at[pl.ds(r0, CROWS), pl.ds(0, LANE)],
                    sems_o[b]),
                pltpu.make_async_copy(
                    valR[b], out_hbm.at[pl.ds(r0, CROWS), pl.ds(LANE, REM)],
                    sems_o[b]),
            )

        for cp in in_cps(0, 0):
            cp.start()

        n_full, last = divmod(SLICE, STG)
        for j in range(n_full + (1 if last else 0)):
            ln = STG if j < n_full else last
            off = sid * SLICE + j * STG
            pltpu.sync_copy(vocab_hbm.at[pl.ds(off, ln)], stg.at[pl.ds(0, ln)])
            pltpu.sync_copy(stg.at[pl.ds(0, ln)], table_sh.at[pl.ds(off, ln)])

        @pl.when(sid == 0)
        def _():
            off = NS * SLICE
            pltpu.sync_copy(vocab_hbm.at[pl.ds(off, TAIL)],
                            stg.at[pl.ds(0, TAIL)])
            pltpu.sync_copy(stg.at[pl.ds(0, TAIL)],
                            table_sh.at[pl.ds(off, TAIL)])

        plsc.subcore_barrier()

        pending_out = [False, False]
        for k in range(NCHUNK):
            b = k % 2
            if k + 1 < NCHUNK:
                for cp in in_cps(k + 1, 1 - b):
                    cp.start()
            for cp in in_cps(k, b):
                cp.wait()
            if pending_out[b]:
                for cp in out_cps(k - 2, b):
                    cp.wait()

            @pl.loop(0, CROWS, step=GROUP)
            def _(g, b=b):
                for u in range(GROUP):
                    pltpu.make_async_copy(
                        table_sh.at[idxL[b].at[g + u]],
                        valL[b].at[g + u], sem_g).start()
                    pltpu.make_async_copy(
                        table_sh.at[idxR[b].at[g + u]],
                        valR[b].at[g + u], sem_g).start()
                for u in range(GROUP):
                    pltpu.make_async_copy(
                        table_sh.at[idxL[b].at[g + u]],
                        valL[b].at[g + u], sem_g).wait()
                    pltpu.make_async_copy(
                        table_sh.at[idxR[b].at[g + u]],
                        valR[b].at[g + u], sem_g).wait()

            for cp in out_cps(k, b):
                cp.start()
            pending_out[b] = True
        for b in range(2):
            if pending_out[b]:
                for cp in out_cps(NCHUNK - 2 + b, b):
                    cp.wait()

    return gather_kernel


_gather = _make_kernel()


def kernel(tokens, vocab_map):
    return _gather(tokens, vocab_map)


# speedup vs baseline: 409.2576x; 1.2598x over previous
import functools

import jax
import jax.numpy as jnp
from jax import lax
from jax.experimental import pallas as pl
from jax.experimental.pallas import tpu as pltpu
from jax.experimental.pallas import tpu_sc as plsc

BATCH = 16384
SEQ_LEN = 200
VOCAB = 1000000
LANE = 128

_info = plsc.get_sparse_core_info()
NC = _info.num_cores
NS = _info.num_subcores
NW = NC * NS
COLS_PER_W = BATCH // NW
CROWS = 40
NRG = SEQ_LEN // CROWS
NCB = COLS_PER_W // LANE
NCHUNK = NRG * NCB
GROUP = 8
STG = 12800
SLICE = (VOCAB // NS) // 8 * 8
TAIL = VOCAB - NS * SLICE


def _make_kernel():
    mesh = plsc.VectorSubcoreMesh(core_axis_name="c", subcore_axis_name="s")

    @functools.partial(
        pl.kernel,
        mesh=mesh,
        out_type=jax.ShapeDtypeStruct((SEQ_LEN, BATCH), jnp.int32),
        scratch_types=[
            pltpu.VMEM_SHARED((VOCAB,), jnp.int32),
            pltpu.VMEM((STG,), jnp.int32),
            [pltpu.VMEM((CROWS, LANE), jnp.int32) for _ in range(2)],
            [pltpu.VMEM((CROWS, LANE), jnp.int32) for _ in range(2)],
            [pltpu.SemaphoreType.DMA for _ in range(2)],
            [pltpu.SemaphoreType.DMA for _ in range(2)],
            pltpu.SemaphoreType.DMA,
        ],
    )
    def gather_kernel(tok_hbm, vocab_hbm, out_hbm, table_sh, stg,
                      idx, val, sems_i, sems_o, sem_g):
        sid = lax.axis_index("s")
        wid = sid * NC + lax.axis_index("c")
        col_base = wid * COLS_PER_W

        def chunk_slice(ref, k):
            rg, cb = k % NRG, k // NRG
            return ref.at[pl.ds(rg * CROWS, CROWS),
                          pl.ds(col_base + cb * LANE, LANE)]

        def in_cp(k, b):
            return pltpu.make_async_copy(chunk_slice(tok_hbm, k), idx[b],
                                         sems_i[b])

        def out_cp(k, b):
            return pltpu.make_async_copy(val[b], chunk_slice(out_hbm, k),
                                         sems_o[b])

        in_cp(0, 0).start()

        n_full, last = divmod(SLICE, STG)
        for j in range(n_full + (1 if last else 0)):
            ln = STG if j < n_full else last
            off = sid * SLICE + j * STG
            pltpu.sync_copy(vocab_hbm.at[pl.ds(off, ln)], stg.at[pl.ds(0, ln)])
            pltpu.sync_copy(stg.at[pl.ds(0, ln)], table_sh.at[pl.ds(off, ln)])

        @pl.when(sid == 0)
        def _():
            off = NS * SLICE
            pltpu.sync_copy(vocab_hbm.at[pl.ds(off, TAIL)],
                            stg.at[pl.ds(0, TAIL)])
            pltpu.sync_copy(stg.at[pl.ds(0, TAIL)],
                            table_sh.at[pl.ds(off, TAIL)])

        plsc.subcore_barrier()

        pending_out = [False, False]
        for k in range(NCHUNK):
            b = k % 2
            if k + 1 < NCHUNK:
                in_cp(k + 1, 1 - b).start()
            in_cp(k, b).wait()
            if pending_out[b]:
                out_cp(k - 2, b).wait()

            @pl.loop(0, CROWS, step=GROUP)
            def _(g, b=b):
                for u in range(GROUP):
                    pltpu.make_async_copy(
                        table_sh.at[idx[b].at[g + u]],
                        val[b].at[g + u], sem_g).start()
                for u in range(GROUP):
                    pltpu.make_async_copy(
                        table_sh.at[idx[b].at[g + u]],
                        val[b].at[g + u], sem_g).wait()

            out_cp(k, b).start()
            pending_out[b] = True
        for b in range(2):
            if pending_out[b]:
                out_cp(NCHUNK - 2 + b, b).wait()

    return gather_kernel


_gather = _make_kernel()


def kernel(tokens, vocab_map):
    return _gather(tokens.T, vocab_map).T
